# baseline (speedup 1.0000x reference)
import jax
import jax.numpy as jnp
from jax import lax
from jax.experimental import pallas as pl
from jax.experimental.pallas import tpu as pltpu
from jax.experimental.pallas import tpu_sc as plsc

N = 10000
D = 128
HD = D // 2
CHUNK = 128
NCORES = 2
NSUB = 16
NPAD = 10240
ZROWS = NPAD // NSUB
IDXB = 8
NCHUNKS = 160
NB = NCHUNKS // IDXB


def _sc_agg(with_counts: bool):
    mesh = plsc.VectorSubcoreMesh(core_axis_name="c", subcore_axis_name="s")
    DEPTH = 4
    SLOTS = 2
    PREFJ = 3
    F = DEPTH // 2
    out_type = [jax.ShapeDtypeStruct((NCORES, NPAD, HD), jnp.float32)]
    if with_counts:
        out_type.append(jax.ShapeDtypeStruct((NPAD,), jnp.float32))
        out_type.append(jax.ShapeDtypeStruct((NSUB, NPAD), jnp.float32))
    scratch = [
        pltpu.VMEM((SLOTS, IDXB * CHUNK), jnp.int32),
        pltpu.VMEM((SLOTS, IDXB, CHUNK), jnp.int32),
    ] + [pltpu.VMEM((CHUNK, HD), jnp.float32) for _ in range(DEPTH)] + [
        pltpu.VMEM((128, HD), jnp.float32),
        pltpu.VMEM((NPAD,), jnp.float32),
        pltpu.VMEM((NSUB, ZROWS), jnp.float32),
        pltpu.VMEM((ZROWS,), jnp.float32),
        pltpu.VMEM_SHARED((NPAD, HD), jnp.float32),
    ] + [pltpu.SemaphoreType.DMA for _ in range(2 * DEPTH + 1)]

    def body(y_hbm, src_hbm, dst_hbm, *rest):
        if with_counts:
            psum_hbm, pcnt_hbm, cshr = rest[0], rest[1], rest[2]
            rest = rest[3:]
        else:
            psum_hbm, pcnt_hbm, cshr = rest[0], None, None
            rest = rest[1:]
        srcb, dstb = rest[0], rest[1]
        rows = rest[2:2 + DEPTH]
        zbuf, cntv, ctile, cmrg, acc = rest[2 + DEPTH:7 + DEPTH]
        sg = rest[7 + DEPTH:7 + 2 * DEPTH]
        ss = rest[7 + 2 * DEPTH:7 + 3 * DEPTH]
        sb = rest[7 + 3 * DEPTH]

        c = lax.axis_index("c")
        s = lax.axis_index("s")

        def fill_row(i, carry):
            for j in range(HD // 16):
                zbuf[i, pl.ds(j * 16, 16)] = jnp.zeros((16,), jnp.float32)
            return carry
        lax.fori_loop(0, 128, fill_row, 0)
        if with_counts:
            def zc(i, carry):
                cntv[pl.ds(i * 16, 16)] = jnp.zeros((16,), jnp.float32)
                return carry
            lax.fori_loop(0, NPAD // 16, zc, 0)

        zbase = s * ZROWS
        for k in range(ZROWS // 128):
            pltpu.sync_copy(zbuf, acc.at[pl.ds(zbase + k * 128, 128)])
        plsc.subcore_barrier()

        dbase = s * NCHUNKS

        def load_block(k, slot):
            pltpu.async_copy(
                src_hbm.at[c, pl.ds(s * NCHUNKS * CHUNK + k * IDXB * CHUNK,
                                    IDXB * CHUNK)],
                srcb.at[slot], sb)
            pltpu.async_copy(dst_hbm.at[pl.ds(dbase + k * IDXB, IDXB)],
                             dstb.at[slot], sb)

        def wait_block(slot):
            pltpu.make_async_copy(src_hbm.at[c, pl.ds(0, IDXB * CHUNK)],
                                  srcb.at[slot], sb).wait()
            pltpu.make_async_copy(dst_hbm.at[pl.ds(0, IDXB)],
                                  dstb.at[slot], sb).wait()

        def start_gather(slot, j, q):
            pltpu.async_copy(y_hbm.at[srcb.at[slot, pl.ds(j * CHUNK, CHUNK)]],
                             rows[q], sg[q])

        def wait_gather(q):
            pltpu.make_async_copy(y_hbm.at[pl.ds(0, CHUNK)], rows[q],
                                  sg[q]).wait()

        def fire_scatter(slot, j, q):
            pltpu.async_copy(rows[q], acc.at[dstb.at[slot, j]], ss[q],
                             add=True)
            if with_counts:
                @pl.when(c == 0)
                def _():
                    ones16 = jnp.ones((16,), jnp.float32)
                    for l in range(CHUNK // 16):
                        idx16 = dstb[slot, j, pl.ds(l * 16, 16)]
                        plsc.addupdate_scatter(cntv, [idx16], ones16)

        def wait_scatter(q):
            pltpu.make_async_copy(rows[q], acc.at[pl.ds(0, CHUNK)],
                                  ss[q]).wait()

        pltpu.sync_copy(src_hbm.at[c, pl.ds(s * NCHUNKS * CHUNK, IDXB * CHUNK)],
                        srcb.at[0])
        pltpu.sync_copy(dst_hbm.at[pl.ds(dbase, IDXB)], dstb.at[0])
        load_block(1, 1)
        for j in range(IDXB):
            if j >= DEPTH:
                wait_scatter(j % DEPTH)
            start_gather(0, j, j % DEPTH)
            if j >= F:
                wait_gather((j - F) % DEPTH)
                fire_scatter(0, j - F, (j - F) % DEPTH)

        def block_body(k, carry):
            slot = lax.rem(k, SLOTS)
            prev = lax.rem(k + SLOTS - 1, SLOTS)
            nxt = lax.rem(k + 1, SLOTS)
            wait_block(slot)
            for j in range(IDXB):
                wait_scatter(j % DEPTH)
                start_gather(slot, j, j % DEPTH)
                wait_gather((j - F) % DEPTH)
                if j < F:
                    fire_scatter(prev, IDXB - F + j, (j - F) % DEPTH)
                else:
                    fire_scatter(slot, j - F, (j - F) % DEPTH)
                if j == PREFJ:
                    @pl.when(k + 1 < NB)
                    def _():
                        load_block(k + 1, nxt)
            return carry
        lax.fori_loop(1, NB, block_body, 0)

        lastslot = (NB - 1) % SLOTS
        for j in range(IDXB - F, IDXB):
            wait_gather(j % DEPTH)
            fire_scatter(lastslot, j, j % DEPTH)
        for q in range(DEPTH):
            wait_scatter(q)
        plsc.subcore_barrier()

        obase = s * ZROWS
        pltpu.sync_copy(acc.at[pl.ds(obase, ZROWS)],
                        psum_hbm.at[c, pl.ds(obase, ZROWS)])
        if with_counts:
            @pl.when(c == 0)
            def _():
                pltpu.sync_copy(cntv, cshr.at[s])
                plsc.subcore_barrier()
                for r in range(NSUB):
                    pltpu.sync_copy(cshr.at[r, pl.ds(obase, ZROWS)],
                                    ctile.at[r])
                def merge(j, carry):
                    sl = pl.ds(j * 16, 16)
                    v = ctile[0, sl]
                    for r in range(1, NSUB):
                        v = v + ctile[r, sl]
                    cmrg[sl] = v
                    return carry
                lax.fori_loop(0, ZROWS // 16, merge, 0)
                pltpu.sync_copy(cmrg, pcnt_hbm.at[pl.ds(obase, ZROWS)])

    return pl.kernel(body, out_type=out_type, mesh=mesh, scratch_types=scratch,
                     compiler_params=pltpu.CompilerParams(
                         use_tc_tiling_on_sc=False,
                         needs_layout_passes=False))


def _tc_layer1(p_ref, x_ref, c_ref, w_ref, b_ref, o_ref):
    ssum = jnp.concatenate([p_ref[0], p_ref[1]], axis=1) + x_ref[...]
    cnt = c_ref[...] + 1.0
    m = ssum / cnt
    h = lax.dot_general(m, w_ref[...], (((1,), (0,)), ((), ())),
                        preferred_element_type=jnp.float32)
    o_ref[...] = jnp.maximum(h + b_ref[...], 0.0)


def _tc_layer2(q_ref, h_ref, c_ref, w2_ref, b2_ref, wc_ref, bc_ref, o_ref,
               acc_ref):
    i = pl.program_id(0)
    ssum = jnp.concatenate([q_ref[0], q_ref[1]], axis=1) + h_ref[...]
    cnt = c_ref[...] + 1.0
    h2 = lax.dot_general(ssum / cnt, w2_ref[...], (((1,), (0,)), ((), ())),
                         preferred_element_type=jnp.float32)
    h2 = jnp.maximum(h2 + b2_ref[...], 0.0)
    part = jnp.sum(h2, axis=0, keepdims=True)

    @pl.when(i == 0)
    def _():
        acc_ref[0:1, :] = part

    @pl.when(i > 0)
    def _():
        acc_ref[0:1, :] = acc_ref[0:1, :] + part

    @pl.when(i == pl.num_programs(0) - 1)
    def _():
        g = jnp.maximum(acc_ref[0:1, :] * (1.0 / N), 0.0)
        o_ref[...] = lax.dot_general(g, wc_ref[...], (((1,), (0,)), ((), ())),
                                     preferred_element_type=jnp.float32) + bc_ref[...]


def kernel(x, edge_index, W1, b1, W2, b2, Wc, bc):
    E = edge_index.shape[1]
    C = Wc.shape[1]
    src = edge_index[0].astype(jnp.int32)
    dst = edge_index[1].astype(jnp.int32)

    EPAD = NSUB * NCHUNKS * CHUNK
    pad = EPAD - E
    if pad:
        ar = jnp.arange(pad, dtype=jnp.int32)
        src = jnp.concatenate([src, (ar * 997) % N])
        dst = jnp.concatenate([dst, N + (ar % (NPAD - N))])
    src2 = jnp.stack([src * 2, src * 2 + 1])
    dst2 = dst.reshape(EPAD // CHUNK, CHUNK)

    agg1 = _sc_agg(with_counts=True)
    agg2 = _sc_agg(with_counts=False)

    psum1, pcnt, _cstage = agg1(x.reshape(2 * N, HD), src2, dst2)
    cnt2 = pcnt.reshape(NPAD, 1)

    BN = 2000
    grid = N // BN
    h1 = pl.pallas_call(
        _tc_layer1,
        grid=(grid,),
        in_specs=[
            pl.BlockSpec((NCORES, BN, HD), lambda i: (0, i, 0)),
            pl.BlockSpec((BN, D), lambda i: (i, 0)),
            pl.BlockSpec((BN, 1), lambda i: (i, 0)),
            pl.BlockSpec((D, D), lambda i: (0, 0)),
            pl.BlockSpec((1, D), lambda i: (0, 0)),
        ],
        out_specs=pl.BlockSpec((BN, D), lambda i: (i, 0)),
        out_shape=jax.ShapeDtypeStruct((N, D), jnp.float32),
    )(psum1, x, cnt2, W1, b1.reshape(1, D))

    (psum2,) = agg2(h1.reshape(2 * N, HD), src2, dst2)

    out = pl.pallas_call(
        _tc_layer2,
        grid=(grid,),
        in_specs=[
            pl.BlockSpec((NCORES, BN, HD), lambda i: (0, i, 0)),
            pl.BlockSpec((BN, D), lambda i: (i, 0)),
            pl.BlockSpec((BN, 1), lambda i: (i, 0)),
            pl.BlockSpec((D, D), lambda i: (0, 0)),
            pl.BlockSpec((1, D), lambda i: (0, 0)),
            pl.BlockSpec((D, C), lambda i: (0, 0)),
            pl.BlockSpec((1, C), lambda i: (0, 0)),
        ],
        out_specs=pl.BlockSpec((1, C), lambda i: (0, 0)),
        out_shape=jax.ShapeDtypeStruct((1, C), jnp.float32),
        scratch_shapes=[pltpu.VMEM((8, D), jnp.float32)],
    )(psum2, h1, cnt2, W2, b2.reshape(1, D), Wc, bc.reshape(1, C))

    return out


# speedup vs baseline: 1.0552x; 1.0552x over previous
import jax
import jax.numpy as jnp
from jax import lax
from jax.experimental import pallas as pl
from jax.experimental.pallas import tpu as pltpu
from jax.experimental.pallas import tpu_sc as plsc

N = 10000
D = 128
HD = D // 2
CHUNK = 128
NCORES = 2
NSUB = 16
NPAD = 10240
ZROWS = NPAD // NSUB
IDXB = 8
NCHUNKS = 160
NB = NCHUNKS // IDXB


def _sc_agg(with_counts: bool):
    mesh = plsc.VectorSubcoreMesh(core_axis_name="c", subcore_axis_name="s")
    DEPTH = 4
    SLOTS = 2
    PREFJ = 3
    F = DEPTH // 2
    out_type = [jax.ShapeDtypeStruct((NCORES, NPAD, HD), jnp.float32)]
    if with_counts:
        out_type.append(jax.ShapeDtypeStruct((NPAD,), jnp.float32))
    scratch = [
        pltpu.VMEM((SLOTS, IDXB * CHUNK), jnp.int32),
        pltpu.VMEM((SLOTS, IDXB, CHUNK), jnp.int32),
    ] + [pltpu.VMEM((CHUNK, HD), jnp.float32) for _ in range(DEPTH)] + [
        pltpu.VMEM((128, HD), jnp.float32),
        pltpu.VMEM((NPAD,), jnp.float32),
        pltpu.VMEM((NSUB, ZROWS), jnp.float32),
        pltpu.VMEM((ZROWS,), jnp.float32),
        pltpu.VMEM_SHARED((NPAD, HD), jnp.float32),
        pltpu.VMEM_SHARED((NSUB, NPAD), jnp.float32),
    ] + [pltpu.SemaphoreType.DMA for _ in range(2 * DEPTH + 1)]

    def body(y_hbm, src_hbm, dst_hbm, *rest):
        if with_counts:
            psum_hbm, pcnt_hbm = rest[0], rest[1]
            rest = rest[2:]
        else:
            psum_hbm, pcnt_hbm = rest[0], None
            rest = rest[1:]
        srcb, dstb = rest[0], rest[1]
        rows = rest[2:2 + DEPTH]
        zbuf, cntv, ctile, cmrg, acc, cshr = rest[2 + DEPTH:8 + DEPTH]
        sg = rest[8 + DEPTH:8 + 2 * DEPTH]
        ss = rest[8 + 2 * DEPTH:8 + 3 * DEPTH]
        sb = rest[8 + 3 * DEPTH]

        c = lax.axis_index("c")
        s = lax.axis_index("s")

        def fill_row(i, carry):
            for j in range(HD // 16):
                zbuf[i, pl.ds(j * 16, 16)] = jnp.zeros((16,), jnp.float32)
            return carry
        lax.fori_loop(0, 128, fill_row, 0)
        if with_counts:
            def zc(i, carry):
                cntv[pl.ds(i * 16, 16)] = jnp.zeros((16,), jnp.float32)
                return carry
            lax.fori_loop(0, NPAD // 16, zc, 0)

        zbase = s * ZROWS
        for k in range(ZROWS // 128):
            pltpu.sync_copy(zbuf, acc.at[pl.ds(zbase + k * 128, 128)])
        plsc.subcore_barrier()

        dbase = s * NCHUNKS

        def load_block(k, slot):
            pltpu.async_copy(
                src_hbm.at[c, pl.ds(s * NCHUNKS * CHUNK + k * IDXB * CHUNK,
                                    IDXB * CHUNK)],
                srcb.at[slot], sb)
            pltpu.async_copy(dst_hbm.at[pl.ds(dbase + k * IDXB, IDXB)],
                             dstb.at[slot], sb)

        def wait_block(slot):
            pltpu.make_async_copy(src_hbm.at[c, pl.ds(0, IDXB * CHUNK)],
                                  srcb.at[slot], sb).wait()
            pltpu.make_async_copy(dst_hbm.at[pl.ds(0, IDXB)],
                                  dstb.at[slot], sb).wait()

        def start_gather(slot, j, q):
            pltpu.async_copy(y_hbm.at[srcb.at[slot, pl.ds(j * CHUNK, CHUNK)]],
                             rows[q], sg[q])

        def wait_gather(q):
            pltpu.make_async_copy(y_hbm.at[pl.ds(0, CHUNK)], rows[q],
                                  sg[q]).wait()

        def fire_scatter(slot, j, q):
            pltpu.async_copy(rows[q], acc.at[dstb.at[slot, j]], ss[q],
                             add=True)
            if with_counts:
                @pl.when(c == 0)
                def _():
                    ones16 = jnp.ones((16,), jnp.float32)
                    for l in range(CHUNK // 16):
                        idx16 = dstb[slot, j, pl.ds(l * 16, 16)]
                        plsc.addupdate_scatter(cntv, [idx16], ones16)

        def wait_scatter(q):
            pltpu.make_async_copy(rows[q], acc.at[pl.ds(0, CHUNK)],
                                  ss[q]).wait()

        pltpu.sync_copy(src_hbm.at[c, pl.ds(s * NCHUNKS * CHUNK, IDXB * CHUNK)],
                        srcb.at[0])
        pltpu.sync_copy(dst_hbm.at[pl.ds(dbase, IDXB)], dstb.at[0])
        load_block(1, 1)
        for j in range(IDXB):
            if j >= DEPTH:
                wait_scatter(j % DEPTH)
            start_gather(0, j, j % DEPTH)
            if j >= F:
                wait_gather((j - F) % DEPTH)
                fire_scatter(0, j - F, (j - F) % DEPTH)

        def block_body(k, carry):
            slot = lax.rem(k, SLOTS)
            prev = lax.rem(k + SLOTS - 1, SLOTS)
            nxt = lax.rem(k + 1, SLOTS)
            wait_block(slot)
            for j in range(IDXB):
                wait_scatter(j % DEPTH)
                start_gather(slot, j, j % DEPTH)
                wait_gather((j - F) % DEPTH)
                if j < F:
                    fire_scatter(prev, IDXB - F + j, (j - F) % DEPTH)
                else:
                    fire_scatter(slot, j - F, (j - F) % DEPTH)
                if j == PREFJ:
                    @pl.when(k + 1 < NB)
                    def _():
                        load_block(k + 1, nxt)
            return carry
        lax.fori_loop(1, NB, block_body, 0)

        lastslot = (NB - 1) % SLOTS
        for j in range(IDXB - F, IDXB):
            wait_gather(j % DEPTH)
            fire_scatter(lastslot, j, j % DEPTH)
        for q in range(DEPTH):
            wait_scatter(q)
        plsc.subcore_barrier()

        obase = s * ZROWS
        pltpu.sync_copy(acc.at[pl.ds(obase, ZROWS)],
                        psum_hbm.at[c, pl.ds(obase, ZROWS)])
        if with_counts:
            @pl.when(c == 0)
            def _():
                pltpu.sync_copy(cntv, cshr.at[s])
                plsc.subcore_barrier()
                for r in range(NSUB):
                    pltpu.sync_copy(cshr.at[r, pl.ds(obase, ZROWS)],
                                    ctile.at[r])
                def merge(j, carry):
                    sl = pl.ds(j * 16, 16)
                    v = ctile[0, sl]
                    for r in range(1, NSUB):
                        v = v + ctile[r, sl]
                    cmrg[sl] = v
                    return carry
                lax.fori_loop(0, ZROWS // 16, merge, 0)
                pltpu.sync_copy(cmrg, pcnt_hbm.at[pl.ds(obase, ZROWS)])

    return pl.kernel(body, out_type=out_type, mesh=mesh, scratch_types=scratch,
                     compiler_params=pltpu.CompilerParams(
                         use_tc_tiling_on_sc=False,
                         needs_layout_passes=False))


def _tc_layer1(p_ref, x_ref, c_ref, w_ref, b_ref, o_ref):
    ssum = jnp.concatenate([p_ref[0], p_ref[1]], axis=1) + x_ref[...]
    cnt = c_ref[...] + 1.0
    m = ssum / cnt
    h = lax.dot_general(m, w_ref[...], (((1,), (0,)), ((), ())),
                        preferred_element_type=jnp.float32)
    o_ref[...] = jnp.maximum(h + b_ref[...], 0.0)


def _tc_layer2(q_ref, h_ref, c_ref, w2_ref, b2_ref, wc_ref, bc_ref, o_ref,
               acc_ref):
    i = pl.program_id(0)
    ssum = jnp.concatenate([q_ref[0], q_ref[1]], axis=1) + h_ref[...]
    cnt = c_ref[...] + 1.0
    h2 = lax.dot_general(ssum / cnt, w2_ref[...], (((1,), (0,)), ((), ())),
                         preferred_element_type=jnp.float32)
    h2 = jnp.maximum(h2 + b2_ref[...], 0.0)
    part = jnp.sum(h2, axis=0, keepdims=True)

    @pl.when(i == 0)
    def _():
        acc_ref[0:1, :] = part

    @pl.when(i > 0)
    def _():
        acc_ref[0:1, :] = acc_ref[0:1, :] + part

    @pl.when(i == pl.num_programs(0) - 1)
    def _():
        g = jnp.maximum(acc_ref[0:1, :] * (1.0 / N), 0.0)
        o_ref[...] = lax.dot_general(g, wc_ref[...], (((1,), (0,)), ((), ())),
                                     preferred_element_type=jnp.float32) + bc_ref[...]


def kernel(x, edge_index, W1, b1, W2, b2, Wc, bc):
    E = edge_index.shape[1]
    C = Wc.shape[1]
    src = edge_index[0].astype(jnp.int32)
    dst = edge_index[1].astype(jnp.int32)

    EPAD = NSUB * NCHUNKS * CHUNK
    pad = EPAD - E
    if pad:
        ar = jnp.arange(pad, dtype=jnp.int32)
        src = jnp.concatenate([src, (ar * 997) % N])
        dst = jnp.concatenate([dst, N + (ar % (NPAD - N))])
    src2 = jnp.stack([src * 2, src * 2 + 1])
    dst2 = dst.reshape(EPAD // CHUNK, CHUNK)

    agg1 = _sc_agg(with_counts=True)
    agg2 = _sc_agg(with_counts=False)

    psum1, pcnt = agg1(x.reshape(2 * N, HD), src2, dst2)
    cnt2 = pcnt.reshape(NPAD, 1)

    BN = 2000
    grid = N // BN
    h1 = pl.pallas_call(
        _tc_layer1,
        grid=(grid,),
        in_specs=[
            pl.BlockSpec((NCORES, BN, HD), lambda i: (0, i, 0)),
            pl.BlockSpec((BN, D), lambda i: (i, 0)),
            pl.BlockSpec((BN, 1), lambda i: (i, 0)),
            pl.BlockSpec((D, D), lambda i: (0, 0)),
            pl.BlockSpec((1, D), lambda i: (0, 0)),
        ],
        out_specs=pl.BlockSpec((BN, D), lambda i: (i, 0)),
        out_shape=jax.ShapeDtypeStruct((N, D), jnp.float32),
    )(psum1, x, cnt2, W1, b1.reshape(1, D))

    (psum2,) = agg2(h1.reshape(2 * N, HD), src2, dst2)

    out = pl.pallas_call(
        _tc_layer2,
        grid=(grid,),
        in_specs=[
            pl.BlockSpec((NCORES, BN, HD), lambda i: (0, i, 0)),
            pl.BlockSpec((BN, D), lambda i: (i, 0)),
            pl.BlockSpec((BN, 1), lambda i: (i, 0)),
            pl.BlockSpec((D, D), lambda i: (0, 0)),
            pl.BlockSpec((1, D), lambda i: (0, 0)),
            pl.BlockSpec((D, C), lambda i: (0, 0)),
            pl.BlockSpec((1, C), lambda i: (0, 0)),
        ],
        out_specs=pl.BlockSpec((1, C), lambda i: (0, 0)),
        out_shape=jax.ShapeDtypeStruct((1, C), jnp.float32),
        scratch_shapes=[pltpu.VMEM((8, D), jnp.float32)],
    )(psum2, h1, cnt2, W2, b2.reshape(1, D), Wc, bc.reshape(1, C))

    return out


# speedup vs baseline: 1.1300x; 1.0709x over previous
import jax
import jax.numpy as jnp
from jax import lax
from jax.experimental import pallas as pl
from jax.experimental.pallas import tpu as pltpu
from jax.experimental.pallas import tpu_sc as plsc

N = 10000
D = 128
HD = D // 2
CHUNK = 128
NCORES = 2
NSUB = 16
NPAD = 10240
ZROWS = NPAD // NSUB
IDXB = 8
NCHUNKS = 160
NB = NCHUNKS // IDXB


def _sc_agg(with_counts: bool):
    mesh = plsc.VectorSubcoreMesh(core_axis_name="c", subcore_axis_name="s")
    DEPTH = 4
    SLOTS = 2
    PREFJ = 3
    F = DEPTH // 2
    out_type = [jax.ShapeDtypeStruct((NPAD, D), jnp.float32)]
    if with_counts:
        out_type.append(jax.ShapeDtypeStruct((NPAD,), jnp.float32))
    scratch = [
        pltpu.VMEM((SLOTS, IDXB * CHUNK), jnp.int32),
        pltpu.VMEM((SLOTS, IDXB, CHUNK), jnp.int32),
    ] + [pltpu.VMEM((CHUNK, HD), jnp.float32) for _ in range(DEPTH)] + [
        pltpu.VMEM((128, HD), jnp.float32),
        pltpu.VMEM((NPAD,), jnp.float32),
        pltpu.VMEM((NSUB, ZROWS), jnp.float32),
        pltpu.VMEM((ZROWS,), jnp.float32),
        pltpu.VMEM_SHARED((NPAD, HD), jnp.float32),
        pltpu.VMEM_SHARED((NSUB, NPAD), jnp.float32),
    ] + [pltpu.SemaphoreType.DMA for _ in range(2 * DEPTH + 1)]

    def body(y_hbm, src_hbm, dst_hbm, *rest):
        if with_counts:
            psum_hbm, pcnt_hbm = rest[0], rest[1]
            rest = rest[2:]
        else:
            psum_hbm, pcnt_hbm = rest[0], None
            rest = rest[1:]
        srcb, dstb = rest[0], rest[1]
        rows = rest[2:2 + DEPTH]
        zbuf, cntv, ctile, cmrg, acc, cshr = rest[2 + DEPTH:8 + DEPTH]
        sg = rest[8 + DEPTH:8 + 2 * DEPTH]
        ss = rest[8 + 2 * DEPTH:8 + 3 * DEPTH]
        sb = rest[8 + 3 * DEPTH]

        c = lax.axis_index("c")
        s = lax.axis_index("s")

        def fill_row(i, carry):
            for j in range(HD // 16):
                zbuf[i, pl.ds(j * 16, 16)] = jnp.zeros((16,), jnp.float32)
            return carry
        lax.fori_loop(0, 128, fill_row, 0)
        if with_counts:
            def zc(i, carry):
                cntv[pl.ds(i * 16, 16)] = jnp.zeros((16,), jnp.float32)
                return carry
            lax.fori_loop(0, NPAD // 16, zc, 0)

        zbase = s * ZROWS
        for k in range(ZROWS // 128):
            pltpu.sync_copy(zbuf, acc.at[pl.ds(zbase + k * 128, 128)])
        plsc.subcore_barrier()

        dbase = s * NCHUNKS

        def load_block(k, slot):
            pltpu.async_copy(
                src_hbm.at[c, pl.ds(s * NCHUNKS * CHUNK + k * IDXB * CHUNK,
                                    IDXB * CHUNK)],
                srcb.at[slot], sb)
            pltpu.async_copy(dst_hbm.at[pl.ds(dbase + k * IDXB, IDXB)],
                             dstb.at[slot], sb)

        def wait_block(slot):
            pltpu.make_async_copy(src_hbm.at[c, pl.ds(0, IDXB * CHUNK)],
                                  srcb.at[slot], sb).wait()
            pltpu.make_async_copy(dst_hbm.at[pl.ds(0, IDXB)],
                                  dstb.at[slot], sb).wait()

        def start_gather(slot, j, q):
            pltpu.async_copy(y_hbm.at[srcb.at[slot, pl.ds(j * CHUNK, CHUNK)]],
                             rows[q], sg[q])

        def wait_gather(q):
            pltpu.make_async_copy(y_hbm.at[pl.ds(0, CHUNK)], rows[q],
                                  sg[q]).wait()

        def fire_scatter(slot, j, q):
            pltpu.async_copy(rows[q], acc.at[dstb.at[slot, j]], ss[q],
                             add=True)
            if with_counts:
                @pl.when(c == 0)
                def _():
                    ones16 = jnp.ones((16,), jnp.float32)
                    for l in range(CHUNK // 16):
                        idx16 = dstb[slot, j, pl.ds(l * 16, 16)]
                        plsc.addupdate_scatter(cntv, [idx16], ones16)

        def wait_scatter(q):
            pltpu.make_async_copy(rows[q], acc.at[pl.ds(0, CHUNK)],
                                  ss[q]).wait()

        pltpu.sync_copy(src_hbm.at[c, pl.ds(s * NCHUNKS * CHUNK, IDXB * CHUNK)],
                        srcb.at[0])
        pltpu.sync_copy(dst_hbm.at[pl.ds(dbase, IDXB)], dstb.at[0])
        load_block(1, 1)
        for j in range(IDXB):
            if j >= DEPTH:
                wait_scatter(j % DEPTH)
            start_gather(0, j, j % DEPTH)
            if j >= F:
                wait_gather((j - F) % DEPTH)
                fire_scatter(0, j - F, (j - F) % DEPTH)

        def block_body(k, carry):
            slot = lax.rem(k, SLOTS)
            prev = lax.rem(k + SLOTS - 1, SLOTS)
            nxt = lax.rem(k + 1, SLOTS)
            wait_block(slot)
            for j in range(IDXB):
                wait_scatter(j % DEPTH)
                start_gather(slot, j, j % DEPTH)
                wait_gather((j - F) % DEPTH)
                if j < F:
                    fire_scatter(prev, IDXB - F + j, (j - F) % DEPTH)
                else:
                    fire_scatter(slot, j - F, (j - F) % DEPTH)
                if j == PREFJ:
                    @pl.when(k + 1 < NB)
                    def _():
                        load_block(k + 1, nxt)
            return carry
        lax.fori_loop(1, NB, block_body, 0)

        lastslot = (NB - 1) % SLOTS
        for j in range(IDXB - F, IDXB):
            wait_gather(j % DEPTH)
            fire_scatter(lastslot, j, j % DEPTH)
        for q in range(DEPTH):
            wait_scatter(q)
        plsc.subcore_barrier()

        obase = s * ZROWS
        pltpu.sync_copy(acc.at[pl.ds(obase, ZROWS)],
                        psum_hbm.at[pl.ds(obase, ZROWS), pl.ds(c * HD, HD)])
        if with_counts:
            @pl.when(c == 0)
            def _():
                pltpu.sync_copy(cntv, cshr.at[s])
                plsc.subcore_barrier()
                for r in range(NSUB):
                    pltpu.sync_copy(cshr.at[r, pl.ds(obase, ZROWS)],
                                    ctile.at[r])
                def merge(j, carry):
                    sl = pl.ds(j * 16, 16)
                    v = ctile[0, sl]
                    for r in range(1, NSUB):
                        v = v + ctile[r, sl]
                    cmrg[sl] = v
                    return carry
                lax.fori_loop(0, ZROWS // 16, merge, 0)
                pltpu.sync_copy(cmrg, pcnt_hbm.at[pl.ds(obase, ZROWS)])

    return pl.kernel(body, out_type=out_type, mesh=mesh, scratch_types=scratch,
                     compiler_params=pltpu.CompilerParams(
                         use_tc_tiling_on_sc=False,
                         needs_layout_passes=False))


def _tc_layer1(p_ref, x_ref, c_ref, w_ref, b_ref, o_ref):
    ssum = p_ref[...] + x_ref[...]
    cnt = c_ref[...] + 1.0
    m = ssum / cnt
    h = lax.dot_general(m, w_ref[...], (((1,), (0,)), ((), ())),
                        preferred_element_type=jnp.float32)
    o_ref[...] = jnp.maximum(h + b_ref[...], 0.0)


def _tc_layer2(q_ref, h_ref, c_ref, w2_ref, b2_ref, wc_ref, bc_ref, o_ref,
               acc_ref):
    i = pl.program_id(0)
    ssum = q_ref[...] + h_ref[...]
    cnt = c_ref[...] + 1.0
    h2 = lax.dot_general(ssum / cnt, w2_ref[...], (((1,), (0,)), ((), ())),
                         preferred_element_type=jnp.float32)
    h2 = jnp.maximum(h2 + b2_ref[...], 0.0)
    part = jnp.sum(h2, axis=0, keepdims=True)

    @pl.when(i == 0)
    def _():
        acc_ref[0:1, :] = part

    @pl.when(i > 0)
    def _():
        acc_ref[0:1, :] = acc_ref[0:1, :] + part

    @pl.when(i == pl.num_programs(0) - 1)
    def _():
        g = jnp.maximum(acc_ref[0:1, :] * (1.0 / N), 0.0)
        o_ref[...] = lax.dot_general(g, wc_ref[...], (((1,), (0,)), ((), ())),
                                     preferred_element_type=jnp.float32) + bc_ref[...]


def kernel(x, edge_index, W1, b1, W2, b2, Wc, bc):
    E = edge_index.shape[1]
    C = Wc.shape[1]
    src = edge_index[0].astype(jnp.int32)
    dst = edge_index[1].astype(jnp.int32)

    EPAD = NSUB * NCHUNKS * CHUNK
    pad = EPAD - E
    if pad:
        ar = jnp.arange(pad, dtype=jnp.int32)
        src = jnp.concatenate([src, (ar * 997) % N])
        dst = jnp.concatenate([dst, N + (ar % (NPAD - N))])
    src2 = jnp.stack([src * 2, src * 2 + 1])
    dst2 = dst.reshape(EPAD // CHUNK, CHUNK)

    agg1 = _sc_agg(with_counts=True)
    agg2 = _sc_agg(with_counts=False)

    psum1, pcnt = agg1(x.reshape(2 * N, HD), src2, dst2)
    cnt2 = pcnt.reshape(NPAD, 1)

    BN = 2000
    grid = N // BN
    h1 = pl.pallas_call(
        _tc_layer1,
        grid=(grid,),
        in_specs=[
            pl.BlockSpec((BN, D), lambda i: (i, 0)),
            pl.BlockSpec((BN, D), lambda i: (i, 0)),
            pl.BlockSpec((BN, 1), lambda i: (i, 0)),
            pl.BlockSpec((D, D), lambda i: (0, 0)),
            pl.BlockSpec((1, D), lambda i: (0, 0)),
        ],
        out_specs=pl.BlockSpec((BN, D), lambda i: (i, 0)),
        out_shape=jax.ShapeDtypeStruct((N, D), jnp.float32),
    )(psum1, x, cnt2, W1, b1.reshape(1, D))

    (psum2,) = agg2(h1.reshape(2 * N, HD), src2, dst2)

    out = pl.pallas_call(
        _tc_layer2,
        grid=(grid,),
        in_specs=[
            pl.BlockSpec((BN, D), lambda i: (i, 0)),
            pl.BlockSpec((BN, D), lambda i: (i, 0)),
            pl.BlockSpec((BN, 1), lambda i: (i, 0)),
            pl.BlockSpec((D, D), lambda i: (0, 0)),
            pl.BlockSpec((1, D), lambda i: (0, 0)),
            pl.BlockSpec((D, C), lambda i: (0, 0)),
            pl.BlockSpec((1, C), lambda i: (0, 0)),
        ],
        out_specs=pl.BlockSpec((1, C), lambda i: (0, 0)),
        out_shape=jax.ShapeDtypeStruct((1, C), jnp.float32),
        scratch_shapes=[pltpu.VMEM((8, D), jnp.float32)],
    )(psum2, h1, cnt2, W2, b2.reshape(1, D), Wc, bc.reshape(1, C))

    return out


# speedup vs baseline: 1.1891x; 1.0523x over previous
import jax
import jax.numpy as jnp
from jax import lax
from jax.experimental import pallas as pl
from jax.experimental.pallas import tpu as pltpu
from jax.experimental.pallas import tpu_sc as plsc

N = 10000
D = 128
HD = D // 2
CHUNK = 128
NCORES = 2
NSUB = 16
NPAD = 10240
ZROWS = NPAD // NSUB
IDXB = 8
NCHUNKS = 160
NB = NCHUNKS // IDXB


def _sc_agg(with_counts: bool):
    mesh = plsc.VectorSubcoreMesh(core_axis_name="c", subcore_axis_name="s")
    DEPTH = 4
    SLOTS = 2
    PREFJ = 3
    F = DEPTH // 2
    out_type = [jax.ShapeDtypeStruct((NPAD, D), jnp.float32)]
    if with_counts:
        out_type.append(jax.ShapeDtypeStruct((NPAD,), jnp.float32))
    scratch = [
        pltpu.VMEM((SLOTS, IDXB, CHUNK), jnp.int32),
        pltpu.VMEM((SLOTS, IDXB, CHUNK), jnp.int32),
    ] + [pltpu.VMEM((CHUNK, HD), jnp.float32) for _ in range(DEPTH)] + [
        pltpu.VMEM((128, HD), jnp.float32),
        pltpu.VMEM((NPAD,), jnp.float32),
        pltpu.VMEM((NSUB, ZROWS), jnp.float32),
        pltpu.VMEM((ZROWS,), jnp.float32),
        pltpu.VMEM_SHARED((NPAD, HD), jnp.float32),
        pltpu.VMEM_SHARED((NSUB, NPAD), jnp.float32),
    ] + [pltpu.SemaphoreType.DMA for _ in range(2 * DEPTH + 1)]

    def body(y_hbm, src_hbm, dst_hbm, *rest):
        if with_counts:
            psum_hbm, pcnt_hbm = rest[0], rest[1]
            rest = rest[2:]
        else:
            psum_hbm, pcnt_hbm = rest[0], None
            rest = rest[1:]
        srcb, dstb = rest[0], rest[1]
        rows = rest[2:2 + DEPTH]
        zbuf, cntv, ctile, cmrg, acc, cshr = rest[2 + DEPTH:8 + DEPTH]
        sg = rest[8 + DEPTH:8 + 2 * DEPTH]
        ss = rest[8 + 2 * DEPTH:8 + 3 * DEPTH]
        sb = rest[8 + 3 * DEPTH]

        c = lax.axis_index("c")
        s = lax.axis_index("s")

        def fill_row(i, carry):
            for j in range(HD // 16):
                zbuf[i, pl.ds(j * 16, 16)] = jnp.zeros((16,), jnp.float32)
            return carry
        lax.fori_loop(0, 128, fill_row, 0)
        if with_counts:
            def zc(i, carry):
                cntv[pl.ds(i * 16, 16)] = jnp.zeros((16,), jnp.float32)
                return carry
            lax.fori_loop(0, NPAD // 16, zc, 0)

        zbase = s * ZROWS
        for k in range(ZROWS // 128):
            pltpu.sync_copy(zbuf, acc.at[pl.ds(zbase + k * 128, 128)])
        plsc.subcore_barrier()

        dbase = s * NCHUNKS

        def load_block(k, slot):
            pltpu.async_copy(src_hbm.at[c, pl.ds(dbase + k * IDXB, IDXB)],
                             srcb.at[slot], sb)
            pltpu.async_copy(dst_hbm.at[pl.ds(dbase + k * IDXB, IDXB)],
                             dstb.at[slot], sb)

        def wait_block(slot):
            pltpu.make_async_copy(src_hbm.at[c, pl.ds(0, IDXB)],
                                  srcb.at[slot], sb).wait()
            pltpu.make_async_copy(dst_hbm.at[pl.ds(0, IDXB)],
                                  dstb.at[slot], sb).wait()

        def start_gather(slot, j, q):
            pltpu.async_copy(y_hbm.at[srcb.at[slot, j]], rows[q], sg[q])

        def wait_gather(q):
            pltpu.make_async_copy(y_hbm.at[pl.ds(0, CHUNK)], rows[q],
                                  sg[q]).wait()

        def fire_scatter(slot, j, q):
            pltpu.async_copy(rows[q], acc.at[dstb.at[slot, j]], ss[q],
                             add=True)
            if with_counts:
                @pl.when(c == 0)
                def _():
                    ones16 = jnp.ones((16,), jnp.float32)
                    for l in range(CHUNK // 16):
                        idx16 = dstb[slot, j, pl.ds(l * 16, 16)]
                        plsc.addupdate_scatter(cntv, [idx16], ones16)

        def wait_scatter(q):
            pltpu.make_async_copy(rows[q], acc.at[pl.ds(0, CHUNK)],
                                  ss[q]).wait()

        pltpu.sync_copy(src_hbm.at[c, pl.ds(dbase, IDXB)], srcb.at[0])
        pltpu.sync_copy(dst_hbm.at[pl.ds(dbase, IDXB)], dstb.at[0])
        load_block(1, 1)
        for j in range(IDXB):
            if j >= DEPTH:
                wait_scatter(j % DEPTH)
            start_gather(0, j, j % DEPTH)
            if j >= F:
                wait_gather((j - F) % DEPTH)
                fire_scatter(0, j - F, (j - F) % DEPTH)

        def block_body(k, carry):
            slot = lax.rem(k, SLOTS)
            prev = lax.rem(k + SLOTS - 1, SLOTS)
            nxt = lax.rem(k + 1, SLOTS)
            wait_block(slot)
            for j in range(IDXB):
                wait_scatter(j % DEPTH)
                start_gather(slot, j, j % DEPTH)
                wait_gather((j - F) % DEPTH)
                if j < F:
                    fire_scatter(prev, IDXB - F + j, (j - F) % DEPTH)
                else:
                    fire_scatter(slot, j - F, (j - F) % DEPTH)
                if j == PREFJ:
                    @pl.when(k + 1 < NB)
                    def _():
                        load_block(k + 1, nxt)
            return carry
        lax.fori_loop(1, NB, block_body, 0)

        lastslot = (NB - 1) % SLOTS
        for j in range(IDXB - F, IDXB):
            wait_gather(j % DEPTH)
            fire_scatter(lastslot, j, j % DEPTH)
        for q in range(DEPTH):
            wait_scatter(q)
        plsc.subcore_barrier()

        obase = s * ZROWS
        pltpu.sync_copy(acc.at[pl.ds(obase, ZROWS)],
                        psum_hbm.at[pl.ds(obase, ZROWS), pl.ds(c * HD, HD)])
        if with_counts:
            @pl.when(c == 0)
            def _():
                pltpu.sync_copy(cntv, cshr.at[s])
                plsc.subcore_barrier()
                for r in range(NSUB):
                    pltpu.sync_copy(cshr.at[r, pl.ds(obase, ZROWS)],
                                    ctile.at[r])
                def merge(j, carry):
                    sl = pl.ds(j * 16, 16)
                    v = ctile[0, sl]
                    for r in range(1, NSUB):
                        v = v + ctile[r, sl]
                    cmrg[sl] = v
                    return carry
                lax.fori_loop(0, ZROWS // 16, merge, 0)
                pltpu.sync_copy(cmrg, pcnt_hbm.at[pl.ds(obase, ZROWS)])

    return pl.kernel(body, out_type=out_type, mesh=mesh, scratch_types=scratch,
                     compiler_params=pltpu.CompilerParams(
                         use_tc_tiling_on_sc=False,
                         needs_layout_passes=False))


def _make_tc_prep(E):
    def _tc_prep(e_ref, s2_ref, d2_ref):
        i = pl.program_id(0)
        nr = s2_ref.shape[1]
        v0 = e_ref[0].reshape(nr, CHUNK)
        v1 = e_ref[1].reshape(nr, CHUNK)
        rel = (lax.broadcasted_iota(jnp.int32, (nr, CHUNK), 0) * CHUNK
               + lax.broadcasted_iota(jnp.int32, (nr, CHUNK), 1))
        absid = i * (nr * CHUNK) + rel
        m = absid < E
        sv = jnp.where(m, v0, absid % N)
        dv = jnp.where(m, v1, N + absid % (NPAD - N))
        s2_ref[0] = sv * 2
        s2_ref[1] = sv * 2 + 1
        d2_ref[...] = dv
    return _tc_prep


def _tc_layer1(p_ref, x_ref, c_ref, w_ref, b_ref, o_ref):
    ssum = p_ref[...] + x_ref[...]
    cnt = c_ref[...] + 1.0
    m = ssum / cnt
    h = lax.dot_general(m, w_ref[...], (((1,), (0,)), ((), ())),
                        preferred_element_type=jnp.float32)
    o_ref[...] = jnp.maximum(h + b_ref[...], 0.0)


def _tc_layer2(q_ref, h_ref, c_ref, w2_ref, b2_ref, wc_ref, bc_ref, o_ref,
               acc_ref):
    i = pl.program_id(0)
    ssum = q_ref[...] + h_ref[...]
    cnt = c_ref[...] + 1.0
    h2 = lax.dot_general(ssum / cnt, w2_ref[...], (((1,), (0,)), ((), ())),
                         preferred_element_type=jnp.float32)
    h2 = jnp.maximum(h2 + b2_ref[...], 0.0)
    part = jnp.sum(h2, axis=0, keepdims=True)

    @pl.when(i == 0)
    def _():
        acc_ref[0:1, :] = part

    @pl.when(i > 0)
    def _():
        acc_ref[0:1, :] = acc_ref[0:1, :] + part

    @pl.when(i == pl.num_programs(0) - 1)
    def _():
        g = jnp.maximum(acc_ref[0:1, :] * (1.0 / N), 0.0)
        o_ref[...] = lax.dot_general(g, wc_ref[...], (((1,), (0,)), ((), ())),
                                     preferred_element_type=jnp.float32) + bc_ref[...]


def kernel(x, edge_index, W1, b1, W2, b2, Wc, bc):
    E = edge_index.shape[1]
    C = Wc.shape[1]

    EPAD = NSUB * NCHUNKS * CHUNK
    ROWS = EPAD // CHUNK
    PG = 5
    BR = ROWS // PG
    src2, dst2 = pl.pallas_call(
        _make_tc_prep(E),
        grid=(PG,),
        in_specs=[pl.BlockSpec((2, BR * CHUNK), lambda i: (0, i))],
        out_specs=[pl.BlockSpec((2, BR, CHUNK), lambda i: (0, i, 0)),
                   pl.BlockSpec((BR, CHUNK), lambda i: (i, 0))],
        out_shape=[jax.ShapeDtypeStruct((2, ROWS, CHUNK), jnp.int32),
                   jax.ShapeDtypeStruct((ROWS, CHUNK), jnp.int32)],
    )(edge_index.astype(jnp.int32))

    agg1 = _sc_agg(with_counts=True)
    agg2 = _sc_agg(with_counts=False)

    psum1, pcnt = agg1(x.reshape(2 * N, HD), src2, dst2)
    cnt2 = pcnt.reshape(NPAD, 1)

    BN = 2000
    grid = N // BN
    h1 = pl.pallas_call(
        _tc_layer1,
        grid=(grid,),
        in_specs=[
            pl.BlockSpec((BN, D), lambda i: (i, 0)),
            pl.BlockSpec((BN, D), lambda i: (i, 0)),
            pl.BlockSpec((BN, 1), lambda i: (i, 0)),
            pl.BlockSpec((D, D), lambda i: (0, 0)),
            pl.BlockSpec((1, D), lambda i: (0, 0)),
        ],
        out_specs=pl.BlockSpec((BN, D), lambda i: (i, 0)),
        out_shape=jax.ShapeDtypeStruct((N, D), jnp.float32),
    )(psum1, x, cnt2, W1, b1.reshape(1, D))

    (psum2,) = agg2(h1.reshape(2 * N, HD), src2, dst2)

    out = pl.pallas_call(
        _tc_layer2,
        grid=(grid,),
        in_specs=[
            pl.BlockSpec((BN, D), lambda i: (i, 0)),
            pl.BlockSpec((BN, D), lambda i: (i, 0)),
            pl.BlockSpec((BN, 1), lambda i: (i, 0)),
            pl.BlockSpec((D, D), lambda i: (0, 0)),
            pl.BlockSpec((1, D), lambda i: (0, 0)),
            pl.BlockSpec((D, C), lambda i: (0, 0)),
            pl.BlockSpec((1, C), lambda i: (0, 0)),
        ],
        out_specs=pl.BlockSpec((1, C), lambda i: (0, 0)),
        out_shape=jax.ShapeDtypeStruct((1, C), jnp.float32),
        scratch_shapes=[pltpu.VMEM((8, D), jnp.float32)],
    )(psum2, h1, cnt2, W2, b2.reshape(1, D), Wc, bc.reshape(1, C))

    return out
